# Initial kernel scaffold; baseline (speedup 1.0000x reference)
import jax
import jax.numpy as jnp
from jax.experimental import pallas as pl


def kernel(x, vec, edge_weight, edge_attr, edge_vec, ln_s, ln_b, Wq, bq, Wk, bk, lnq_s, lnq_b, lnk_s, lnk_b, Wv, bv, Wvec, Wdk, bdk, Wdv, bdv, Wo, bo, senders, receivers):
    raise NotImplementedError("write your pallas kernel here")


# speedup vs baseline: 2.3668x; 2.3668x over previous
import functools
import math

import jax
import jax.numpy as jnp
import numpy as np
from jax.experimental import pallas as pl
from jax.experimental.pallas import tpu as pltpu

N = 10000
E = 320000
H = 8
D = 16
HC = 128
NRBF = 32
CUTOFF_UPPER = 5.0

BN = 200
BE = 2000

_EPS = 1e-6


def _head_mask():
    r = jax.lax.broadcasted_iota(jnp.int32, (HC, H), 0) // D
    c = jax.lax.broadcasted_iota(jnp.int32, (HC, H), 1)
    return (r == c).astype(jnp.float32)


def _silu(x):
    return x * jax.nn.sigmoid(x)


def _node_body(x_ref, vec_ref, Wq_ref, bq_ref, Wk_ref, bk_ref, Wv_ref, bv_ref,
               Wvec_ref, lns_ref, lnb_ref, lnqs_ref, lnqb_ref, lnks_ref,
               lnkb_ref, q_ref, k_ref, v_ref, vd_ref, v3_ref):
    xb = x_ref[...]
    mean = jnp.mean(xb, axis=1, keepdims=True)
    xc = xb - mean
    var = jnp.mean(xc * xc, axis=1, keepdims=True)
    xn = xc * jax.lax.rsqrt(var + _EPS) * lns_ref[...] + lnb_ref[...]

    MH = _head_mask()

    def headln(z, s, b):
        m = (z @ MH) * (1.0 / D)
        mb = m @ MH.T
        zc = z - mb
        v2 = ((zc * zc) @ MH) * (1.0 / D)
        vb = v2 @ MH.T
        return zc * jax.lax.rsqrt(vb + _EPS) * s + b

    q_ref[...] = headln(xn @ Wq_ref[...] + bq_ref[...], lnqs_ref[...],
                        lnqb_ref[...])
    k_ref[...] = headln(xn @ Wk_ref[...] + bk_ref[...], lnks_ref[...],
                        lnkb_ref[...])
    v_ref[...] = xn @ Wv_ref[...] + bv_ref[...]

    vecb = vec_ref[...]
    Wvec = Wvec_ref[...]
    acc = jnp.zeros((vecb.shape[0], HC), jnp.float32)
    for c in range(3):
        p = vecb[:, c * HC:(c + 1) * HC] @ Wvec
        acc = acc + p[:, :HC] * p[:, HC:2 * HC]
        v3_ref[:, c * HC:(c + 1) * HC] = p[:, 2 * HC:]
    vd_ref[...] = acc


def _node_stage(x, vecf, Wq, bq, Wk, bk, Wv_p, bv_p, Wvec, ln_s, ln_b,
                lnq_s, lnq_b, lnk_s, lnk_b):
    nb = N // BN
    row = lambda i: (i, 0)
    rep = lambda i: (0, 0)
    out_shapes = (
        jax.ShapeDtypeStruct((N, HC), jnp.float32),
        jax.ShapeDtypeStruct((N, HC), jnp.float32),
        jax.ShapeDtypeStruct((N, 3 * HC), jnp.float32),
        jax.ShapeDtypeStruct((N, HC), jnp.float32),
        jax.ShapeDtypeStruct((N, 3 * HC), jnp.float32),
    )
    in_specs = [
        pl.BlockSpec((BN, HC), row),
        pl.BlockSpec((BN, 3 * HC), row),
        pl.BlockSpec((HC, HC), rep),
        pl.BlockSpec((1, HC), rep),
        pl.BlockSpec((HC, HC), rep),
        pl.BlockSpec((1, HC), rep),
        pl.BlockSpec((HC, 3 * HC), rep),
        pl.BlockSpec((1, 3 * HC), rep),
        pl.BlockSpec((HC, 3 * HC), rep),
        pl.BlockSpec((1, HC), rep),
        pl.BlockSpec((1, HC), rep),
        pl.BlockSpec((1, HC), rep),
        pl.BlockSpec((1, HC), rep),
        pl.BlockSpec((1, HC), rep),
        pl.BlockSpec((1, HC), rep),
    ]
    out_specs = (
        pl.BlockSpec((BN, HC), row),
        pl.BlockSpec((BN, HC), row),
        pl.BlockSpec((BN, 3 * HC), row),
        pl.BlockSpec((BN, HC), row),
        pl.BlockSpec((BN, 3 * HC), row),
    )
    return pl.pallas_call(
        _node_body, grid=(nb,), in_specs=in_specs, out_specs=out_specs,
        out_shape=out_shapes,
    )(x, vecf, Wq, bq, Wk, bk, Wv_p, bv_p, Wvec, ln_s, ln_b, lnq_s, lnq_b,
      lnk_s, lnk_b)


def _msg_body(qi_ref, kj_ref, vj_ref, vecj_ref, ea_ref, evw_ref, Wdk_ref,
              bdk_ref, Wdv_ref, bdv_ref, m_ref):
    ea = ea_ref[...]
    dk = _silu(ea @ Wdk_ref[...] + bdk_ref[...])
    dv = _silu(ea @ Wdv_ref[...] + bdv_ref[...])

    evw = evw_ref[...]
    w = evw[:, 3:4]
    cut = 0.5 * (jnp.cos(w * (math.pi / CUTOFF_UPPER)) + 1.0)
    cut = cut * (w < CUTOFF_UPPER).astype(jnp.float32)

    MH = _head_mask()
    t = qi_ref[...] * kj_ref[...] * dk
    attn = _silu(t @ MH) * cut
    ae = attn @ MH.T

    vj = vj_ref[...]
    xmv = vj[:, :HC] * dv[:, :HC] * ae
    A = vj[:, HC:2 * HC] * dv[:, HC:2 * HC]
    B = vj[:, 2 * HC:] * dv[:, 2 * HC:]
    vecj = vecj_ref[...]
    m_ref[0] = xmv
    for c in range(3):
        m_ref[c + 1] = vecj[:, c * HC:(c + 1) * HC] * A + B * evw[:, c:c + 1]


def _msg_stage(qi, kj, vj, vecj, edge_attr, evw, Wdk, bdk, Wdv_p, bdv_p):
    nb = E // BE
    row = lambda i: (i, 0)
    rep = lambda i: (0, 0)
    in_specs = [
        pl.BlockSpec((BE, HC), row),
        pl.BlockSpec((BE, HC), row),
        pl.BlockSpec((BE, 3 * HC), row),
        pl.BlockSpec((BE, 3 * HC), row),
        pl.BlockSpec((BE, NRBF), row),
        pl.BlockSpec((BE, 4), row),
        pl.BlockSpec((NRBF, HC), rep),
        pl.BlockSpec((1, HC), rep),
        pl.BlockSpec((NRBF, 3 * HC), rep),
        pl.BlockSpec((1, 3 * HC), rep),
    ]
    out_spec = pl.BlockSpec((4, BE, HC), lambda i: (0, i, 0))
    return pl.pallas_call(
        _msg_body, grid=(nb,), in_specs=in_specs, out_specs=out_spec,
        out_shape=jax.ShapeDtypeStruct((4, E, HC), jnp.float32),
    )(qi, kj, vj, vecj, edge_attr, evw, Wdk, bdk, Wdv_p, bdv_p)


def _out_body(s_ref, vd_ref, v3_ref, Wo_ref, bo_ref, dx_ref, dvec_ref):
    s = s_ref[...]
    o = s[0] @ Wo_ref[...] + bo_ref[...]
    o1, o2, o3 = o[:, :HC], o[:, HC:2 * HC], o[:, 2 * HC:]
    dx_ref[...] = vd_ref[...] * o2 + o3
    v3 = v3_ref[...]
    for c in range(3):
        dvec_ref[:, c * HC:(c + 1) * HC] = v3[:, c * HC:(c + 1) * HC] * o1 \
            + s[c + 1]


def _out_stage(S, vec_dot, vec3, Wo, bo):
    nb = N // BN
    row = lambda i: (i, 0)
    rep = lambda i: (0, 0)
    in_specs = [
        pl.BlockSpec((4, BN, HC), lambda i: (0, i, 0)),
        pl.BlockSpec((BN, HC), row),
        pl.BlockSpec((BN, 3 * HC), row),
        pl.BlockSpec((HC, 3 * HC), rep),
        pl.BlockSpec((1, 3 * HC), rep),
    ]
    out_specs = (
        pl.BlockSpec((BN, HC), row),
        pl.BlockSpec((BN, 3 * HC), row),
    )
    return pl.pallas_call(
        _out_body, grid=(nb,), in_specs=in_specs, out_specs=out_specs,
        out_shape=(jax.ShapeDtypeStruct((N, HC), jnp.float32),
                   jax.ShapeDtypeStruct((N, 3 * HC), jnp.float32)),
    )(S, vec_dot, vec3, Wo, bo)


def _vperm():
    perm = np.empty((3 * HC,), np.int32)
    for p in range(3):
        for h in range(H):
            for d in range(D):
                perm[p * HC + h * D + d] = h * 3 * D + p * D + d
    return perm


def kernel(x, vec, edge_weight, edge_attr, edge_vec, ln_s, ln_b, Wq, bq, Wk,
           bk, lnq_s, lnq_b, lnk_s, lnk_b, Wv, bv, Wvec, Wdk, bdk, Wdv, bdv,
           Wo, bo, senders, receivers):
    perm = _vperm()
    Wv_p = Wv[:, perm]
    bv_p = bv[perm].reshape(1, -1)
    Wdv_p = Wdv[:, perm]
    bdv_p = bdv[perm].reshape(1, -1)
    vecf = vec.reshape(N, 3 * HC)
    evw = jnp.concatenate([edge_vec, edge_weight], axis=1)
    r1 = lambda a: a.reshape(1, -1)
    tile8 = lambda a: jnp.tile(a, (H,)).reshape(1, -1)

    q, k, v, vec_dot, vec3 = _node_stage(
        x, vecf, Wq, r1(bq), Wk, r1(bk), Wv_p, bv_p, Wvec, r1(ln_s),
        r1(ln_b), tile8(lnq_s), tile8(lnq_b), tile8(lnk_s), tile8(lnk_b))

    qi = q[receivers]
    kj = k[senders]
    vj = v[senders]
    vecj = vecf[senders]

    M = _msg_stage(qi, kj, vj, vecj, edge_attr, evw, Wdk, r1(bdk), Wdv_p,
                   bdv_p)

    S = jax.vmap(
        lambda m: jax.ops.segment_sum(m, receivers, num_segments=N))(M)

    dx, dvecf = _out_stage(S, vec_dot, vec3, Wo, r1(bo))
    return (dx, dvecf.reshape(N, 3, HC))


# speedup vs baseline: 25.7433x; 10.8769x over previous
import functools
import math

import jax
import jax.numpy as jnp
import numpy as np
from jax import lax
from jax.experimental import pallas as pl
from jax.experimental.pallas import tpu as pltpu
from jax.experimental.pallas import tpu_sc as plsc

N = 10000
E = 320000
H = 8
D = 16
HC = 128
NRBF = 32
CUTOFF_UPPER = 5.0

BN = 200
BE = 2000

_EPS = 1e-6


def _head_mask():
    r = jax.lax.broadcasted_iota(jnp.int32, (HC, H), 0) // D
    c = jax.lax.broadcasted_iota(jnp.int32, (HC, H), 1)
    return (r == c).astype(jnp.float32)


def _silu(x):
    return x * jax.nn.sigmoid(x)


def _node_body(x_ref, vec_ref, Wq_ref, bq_ref, Wk_ref, bk_ref, Wv_ref, bv_ref,
               Wvec_ref, lns_ref, lnb_ref, lnqs_ref, lnqb_ref, lnks_ref,
               lnkb_ref, q_ref, k_ref, v_ref, vd_ref, v3_ref):
    xb = x_ref[...]
    mean = jnp.mean(xb, axis=1, keepdims=True)
    xc = xb - mean
    var = jnp.mean(xc * xc, axis=1, keepdims=True)
    xn = xc * jax.lax.rsqrt(var + _EPS) * lns_ref[...] + lnb_ref[...]

    MH = _head_mask()

    def headln(z, s, b):
        m = (z @ MH) * (1.0 / D)
        mb = m @ MH.T
        zc = z - mb
        v2 = ((zc * zc) @ MH) * (1.0 / D)
        vb = v2 @ MH.T
        return zc * jax.lax.rsqrt(vb + _EPS) * s + b

    q_ref[...] = headln(xn @ Wq_ref[...] + bq_ref[...], lnqs_ref[...],
                        lnqb_ref[...])
    k_ref[...] = headln(xn @ Wk_ref[...] + bk_ref[...], lnks_ref[...],
                        lnkb_ref[...])
    v_ref[...] = xn @ Wv_ref[...] + bv_ref[...]

    vecb = vec_ref[...]
    Wvec = Wvec_ref[...]
    acc = jnp.zeros((vecb.shape[0], HC), jnp.float32)
    for c in range(3):
        p = vecb[:, c * HC:(c + 1) * HC] @ Wvec
        acc = acc + p[:, :HC] * p[:, HC:2 * HC]
        v3_ref[:, c * HC:(c + 1) * HC] = p[:, 2 * HC:]
    vd_ref[...] = acc


def _node_stage(x, vecf, Wq, bq, Wk, bk, Wv_p, bv_p, Wvec, ln_s, ln_b,
                lnq_s, lnq_b, lnk_s, lnk_b):
    nb = N // BN
    row = lambda i: (i, 0)
    rep = lambda i: (0, 0)
    out_shapes = (
        jax.ShapeDtypeStruct((N, HC), jnp.float32),
        jax.ShapeDtypeStruct((N, HC), jnp.float32),
        jax.ShapeDtypeStruct((N, 3 * HC), jnp.float32),
        jax.ShapeDtypeStruct((N, HC), jnp.float32),
        jax.ShapeDtypeStruct((N, 3 * HC), jnp.float32),
    )
    in_specs = [
        pl.BlockSpec((BN, HC), row),
        pl.BlockSpec((BN, 3 * HC), row),
        pl.BlockSpec((HC, HC), rep),
        pl.BlockSpec((1, HC), rep),
        pl.BlockSpec((HC, HC), rep),
        pl.BlockSpec((1, HC), rep),
        pl.BlockSpec((HC, 3 * HC), rep),
        pl.BlockSpec((1, 3 * HC), rep),
        pl.BlockSpec((HC, 3 * HC), rep),
        pl.BlockSpec((1, HC), rep),
        pl.BlockSpec((1, HC), rep),
        pl.BlockSpec((1, HC), rep),
        pl.BlockSpec((1, HC), rep),
        pl.BlockSpec((1, HC), rep),
        pl.BlockSpec((1, HC), rep),
    ]
    out_specs = (
        pl.BlockSpec((BN, HC), row),
        pl.BlockSpec((BN, HC), row),
        pl.BlockSpec((BN, 3 * HC), row),
        pl.BlockSpec((BN, HC), row),
        pl.BlockSpec((BN, 3 * HC), row),
    )
    return pl.pallas_call(
        _node_body, grid=(nb,), in_specs=in_specs, out_specs=out_specs,
        out_shape=out_shapes,
    )(x, vecf, Wq, bq, Wk, bk, Wv_p, bv_p, Wvec, ln_s, ln_b, lnq_s, lnq_b,
      lnk_s, lnk_b)


def _msg_body(qi_ref, kj_ref, vj_ref, vecj_ref, ea_ref, evw_ref, Wdk_ref,
              bdk_ref, Wdv_ref, bdv_ref, m_ref):
    ea = ea_ref[...]
    dk = _silu(ea @ Wdk_ref[...] + bdk_ref[...])
    dv = _silu(ea @ Wdv_ref[...] + bdv_ref[...])

    evw = evw_ref[...]
    w = evw[:, 3:4]
    cut = 0.5 * (jnp.cos(w * (math.pi / CUTOFF_UPPER)) + 1.0)
    cut = cut * (w < CUTOFF_UPPER).astype(jnp.float32)

    MH = _head_mask()
    t = qi_ref[...] * kj_ref[...] * dk
    attn = _silu(t @ MH) * cut
    ae = attn @ MH.T

    vj = vj_ref[...]
    xmv = vj[:, :HC] * dv[:, :HC] * ae
    A = vj[:, HC:2 * HC] * dv[:, HC:2 * HC]
    B = vj[:, 2 * HC:] * dv[:, 2 * HC:]
    vecj = vecj_ref[...]
    m_ref[0] = xmv
    for c in range(3):
        m_ref[c + 1] = vecj[:, c * HC:(c + 1) * HC] * A + B * evw[:, c:c + 1]


def _msg_stage(qi, kj, vj, vecj, edge_attr, evw, Wdk, bdk, Wdv_p, bdv_p):
    nb = E // BE
    row = lambda i: (i, 0)
    rep = lambda i: (0, 0)
    in_specs = [
        pl.BlockSpec((BE, HC), row),
        pl.BlockSpec((BE, HC), row),
        pl.BlockSpec((BE, 3 * HC), row),
        pl.BlockSpec((BE, 3 * HC), row),
        pl.BlockSpec((BE, NRBF), row),
        pl.BlockSpec((BE, 4), row),
        pl.BlockSpec((NRBF, HC), rep),
        pl.BlockSpec((1, HC), rep),
        pl.BlockSpec((NRBF, 3 * HC), rep),
        pl.BlockSpec((1, 3 * HC), rep),
    ]
    out_spec = pl.BlockSpec((4, BE, HC), lambda i: (0, i, 0))
    return pl.pallas_call(
        _msg_body, grid=(nb,), in_specs=in_specs, out_specs=out_spec,
        out_shape=jax.ShapeDtypeStruct((4, E, HC), jnp.float32),
    )(qi, kj, vj, vecj, edge_attr, evw, Wdk, bdk, Wdv_p, bdv_p)


def _out_body(s_ref, vd_ref, v3_ref, Wo_ref, bo_ref, dx_ref, dvec_ref):
    s = s_ref[...]
    o = s[0] @ Wo_ref[...] + bo_ref[...]
    o1, o2, o3 = o[:, :HC], o[:, HC:2 * HC], o[:, 2 * HC:]
    dx_ref[...] = vd_ref[...] * o2 + o3
    v3 = v3_ref[...]
    for c in range(3):
        dvec_ref[:, c * HC:(c + 1) * HC] = v3[:, c * HC:(c + 1) * HC] * o1 \
            + s[c + 1]


def _out_stage(S, vec_dot, vec3, Wo, bo):
    nb = N // BN
    row = lambda i: (i, 0)
    rep = lambda i: (0, 0)
    in_specs = [
        pl.BlockSpec((4, BN, HC), lambda i: (0, i, 0)),
        pl.BlockSpec((BN, HC), row),
        pl.BlockSpec((BN, 3 * HC), row),
        pl.BlockSpec((HC, 3 * HC), rep),
        pl.BlockSpec((1, 3 * HC), rep),
    ]
    out_specs = (
        pl.BlockSpec((BN, HC), row),
        pl.BlockSpec((BN, 3 * HC), row),
    )
    return pl.pallas_call(
        _out_body, grid=(nb,), in_specs=in_specs, out_specs=out_specs,
        out_shape=(jax.ShapeDtypeStruct((N, HC), jnp.float32),
                   jax.ShapeDtypeStruct((N, 3 * HC), jnp.float32)),
    )(S, vec_dot, vec3, Wo, bo)


NC = 2
NS = 16
NW = NC * NS
GCH = 80


def _sc_gather(q, k, v, vecf, senders, receivers):
    epw = E // NW
    nch = epw // GCH
    mesh = plsc.VectorSubcoreMesh(core_axis_name="c", subcore_axis_name="s")

    @functools.partial(
        pl.kernel, mesh=mesh,
        out_type=(
            jax.ShapeDtypeStruct((E, HC), jnp.float32),
            jax.ShapeDtypeStruct((E, HC), jnp.float32),
            jax.ShapeDtypeStruct((E, 3 * HC), jnp.float32),
            jax.ShapeDtypeStruct((E, 3 * HC), jnp.float32),
        ),
        scratch_types=[
            pltpu.VMEM((GCH,), jnp.int32),
            pltpu.VMEM((GCH,), jnp.int32),
            pltpu.VMEM((GCH, HC), jnp.float32),
            pltpu.VMEM((GCH, 3 * HC), jnp.float32),
            pltpu.SemaphoreType.DMA,
        ],
    )
    def gk(q_hbm, k_hbm, v_hbm, vec_hbm, send_hbm, recv_hbm,
           qi_hbm, kj_hbm, vj_hbm, vecj_hbm, idx_r, idx_s, rows_a, rows_b,
           sem):
        wid = lax.axis_index("s") * NC + lax.axis_index("c")
        base0 = wid * epw

        def body(i, _):
            b = pl.multiple_of(base0 + i * GCH, 8)
            pltpu.sync_copy(recv_hbm.at[pl.ds(b, GCH)], idx_r)
            pltpu.sync_copy(send_hbm.at[pl.ds(b, GCH)], idx_s)
            pltpu.async_copy(q_hbm.at[idx_r], rows_a, sem).wait()
            pltpu.sync_copy(rows_a, qi_hbm.at[pl.ds(b, GCH)])
            pltpu.async_copy(k_hbm.at[idx_s], rows_a, sem).wait()
            pltpu.sync_copy(rows_a, kj_hbm.at[pl.ds(b, GCH)])
            pltpu.async_copy(v_hbm.at[idx_s], rows_b, sem).wait()
            pltpu.sync_copy(rows_b, vj_hbm.at[pl.ds(b, GCH)])
            pltpu.async_copy(vec_hbm.at[idx_s], rows_b, sem).wait()
            pltpu.sync_copy(rows_b, vecj_hbm.at[pl.ds(b, GCH)])
            return ()

        lax.fori_loop(0, nch, body, (), unroll=False)

    return gk(q, k, v, vecf, senders, receivers)


SCH = 80
NPAD = 10240
NPT = NPAD // NS


def _sc_scatter(m_flat, receivers, zeros):
    ept = E // NS
    nch = ept // SCH
    mesh = plsc.VectorSubcoreMesh(core_axis_name="c", subcore_axis_name="s")

    @functools.partial(
        pl.kernel, mesh=mesh,
        out_type=jax.ShapeDtypeStruct((4 * NPAD, HC), jnp.float32),
        scratch_types=[
            pltpu.VMEM_SHARED((NPAD, HC), jnp.float32),
            pltpu.VMEM((SCH,), jnp.int32),
            pltpu.VMEM((SCH, HC), jnp.float32),
        ],
    )
    def sk(m_hbm, recv_hbm, z_hbm, s_hbm, shared, idx, rows):
        cid = lax.axis_index("c")
        sid = lax.axis_index("s")
        r0 = sid * NPT
        e0 = sid * ept
        for j in range(2):
            ch = cid * 2 + j
            pltpu.sync_copy(z_hbm, shared.at[pl.ds(r0, NPT)])
            plsc.subcore_barrier()

            def body(i, _):
                b = pl.multiple_of(e0 + i * SCH, 8)
                mb = pl.multiple_of(ch * E + b, 8)
                pltpu.sync_copy(recv_hbm.at[pl.ds(b, SCH)], idx)
                pltpu.sync_copy(m_hbm.at[pl.ds(mb, SCH)], rows)
                pltpu.sync_copy(rows, shared.at[idx], add=True)
                return ()

            lax.fori_loop(0, nch, body, (), unroll=False)
            plsc.subcore_barrier()
            pltpu.sync_copy(shared.at[pl.ds(r0, NPT)],
                            s_hbm.at[pl.ds(ch * NPAD + r0, NPT)])
            plsc.subcore_barrier()

    return sk(m_flat, receivers, zeros)


def _vperm():
    perm = np.empty((3 * HC,), np.int32)
    for p in range(3):
        for h in range(H):
            for d in range(D):
                perm[p * HC + h * D + d] = h * 3 * D + p * D + d
    return perm


def kernel(x, vec, edge_weight, edge_attr, edge_vec, ln_s, ln_b, Wq, bq, Wk,
           bk, lnq_s, lnq_b, lnk_s, lnk_b, Wv, bv, Wvec, Wdk, bdk, Wdv, bdv,
           Wo, bo, senders, receivers):
    perm = _vperm()
    Wv_p = Wv[:, perm]
    bv_p = bv[perm].reshape(1, -1)
    Wdv_p = Wdv[:, perm]
    bdv_p = bdv[perm].reshape(1, -1)
    vecf = vec.reshape(N, 3 * HC)
    evw = jnp.concatenate([edge_vec, edge_weight], axis=1)
    r1 = lambda a: a.reshape(1, -1)
    tile8 = lambda a: jnp.tile(a, (H,)).reshape(1, -1)

    q, k, v, vec_dot, vec3 = _node_stage(
        x, vecf, Wq, r1(bq), Wk, r1(bk), Wv_p, bv_p, Wvec, r1(ln_s),
        r1(ln_b), tile8(lnq_s), tile8(lnq_b), tile8(lnk_s), tile8(lnk_b))

    qi, kj, vj, vecj = _sc_gather(q, k, v, vecf, senders, receivers)

    M = _msg_stage(qi, kj, vj, vecj, edge_attr, evw, Wdk, r1(bdk), Wdv_p,
                   bdv_p)

    zeros = jnp.zeros((NPT, HC), jnp.float32)
    S = _sc_scatter(M.reshape(4 * E, HC), receivers,
                    zeros).reshape(4, NPAD, HC)

    dx, dvecf = _out_stage(S, vec_dot, vec3, Wo, r1(bo))
    return (dx, dvecf.reshape(N, 3, HC))
